# baseline (speedup 1.0000x reference)
import jax
import jax.numpy as jnp
from jax.experimental import pallas as pl

_NUM_RANKS = 1024
_MAX_TOK = 77
_D = 768
_CTX = 16
_TPR = 4
_RB = 16


def _body(ctx_ref, rank_ref, sent_ref, out_ref):
    sent = sent_ref[...]
    ctx = jnp.broadcast_to(ctx_ref[...][None], (_RB, _CTX, _D))
    out_ref[...] = jnp.concatenate(
        [sent[:, :1, :], ctx, rank_ref[...], sent[:, 1 + _CTX + _TPR:, :]],
        axis=1,
    )


def kernel(context_embeds, rank_embeds, sentence_embeds):
    return pl.pallas_call(
        _body,
        grid=(_NUM_RANKS // _RB,),
        in_specs=[
            pl.BlockSpec((_CTX, _D), lambda i: (0, 0)),
            pl.BlockSpec((_RB, _TPR, _D), lambda i: (i, 0, 0)),
            pl.BlockSpec((_RB, _MAX_TOK, _D), lambda i: (i, 0, 0)),
        ],
        out_specs=pl.BlockSpec((_RB, _MAX_TOK, _D), lambda i: (i, 0, 0)),
        out_shape=jax.ShapeDtypeStruct((_NUM_RANKS, _MAX_TOK, _D), jnp.float32),
    )(context_embeds, rank_embeds, sentence_embeds)
